# Initial kernel scaffold; baseline (speedup 1.0000x reference)
import functools

import jax
import jax.numpy as jnp
from jax import lax
from jax.experimental import pallas as pl
from jax.experimental.pallas import tpu as pltpu
from jax.experimental.pallas import tpu_sc as plsc

_NUM_CORES = 2
_NUM_SUBCORES = 16
_NW = _NUM_CORES * _NUM_SUBCORES
_CHUNK = 512


@functools.partial(jax.jit, static_argnums=(2, 3))
def _gather_flat(idx, table, n, d):
    per_w = n // _NW
    n_chunks = per_w // _CHUNK

    @functools.partial(
        pl.kernel,
        out_type=jax.ShapeDtypeStruct((n, d), jnp.float32),
        mesh=plsc.VectorSubcoreMesh(core_axis_name="c", subcore_axis_name="s"),
        scratch_types=[
            pltpu.VMEM((_CHUNK,), jnp.int32),
            pltpu.VMEM((_CHUNK, d), jnp.float32),
            pltpu.SemaphoreType.DMA,
        ],
    )
    def _impl(idx_hbm, table_hbm, out_hbm, idx_v, rows_v, sem):
        wid = lax.axis_index("s") * _NUM_CORES + lax.axis_index("c")
        base = wid * per_w

        def body(c, carry):
            off = base + c * _CHUNK
            pltpu.sync_copy(idx_hbm.at[pl.ds(off, _CHUNK)], idx_v)
            pltpu.async_copy(table_hbm.at[idx_v], rows_v, sem).wait()
            pltpu.sync_copy(rows_v, out_hbm.at[pl.ds(off, _CHUNK)])
            return carry

        lax.fori_loop(0, n_chunks, body, 0)

    return _impl(idx, table)


def kernel(x, table):
    b, l = x.shape
    v, d = table.shape
    n = b * l
    out = _gather_flat(x.reshape(n), table, n, d)
    return out.reshape(b, l, d)


# speedup vs baseline: 1.0740x; 1.0740x over previous
import functools

import jax
import jax.numpy as jnp
from jax import lax
from jax.experimental import pallas as pl
from jax.experimental.pallas import tpu as pltpu
from jax.experimental.pallas import tpu_sc as plsc

_NUM_CORES = 2
_NUM_SUBCORES = 16
_NW = _NUM_CORES * _NUM_SUBCORES
_CHUNK = 512


@functools.partial(jax.jit, static_argnums=(2, 3))
def _gather_flat(idx, table, n, d):
    per_w = n // _NW
    n_chunks = per_w // _CHUNK

    @functools.partial(
        pl.kernel,
        out_type=jax.ShapeDtypeStruct((n, d), jnp.float32),
        mesh=plsc.VectorSubcoreMesh(core_axis_name="c", subcore_axis_name="s"),
        scratch_types=[
            pltpu.VMEM((_CHUNK,), jnp.int32),
            pltpu.VMEM((_CHUNK, d), jnp.float32),
            pltpu.SemaphoreType.DMA,
        ],
        compiler_params=pltpu.CompilerParams(use_tc_tiling_on_sc=False),
    )
    def _impl(idx_hbm, table_hbm, out_hbm, idx_v, rows_v, sem):
        wid = lax.axis_index("s") * _NUM_CORES + lax.axis_index("c")
        base = wid * per_w

        def body(c, carry):
            off = base + c * _CHUNK
            pltpu.sync_copy(idx_hbm.at[pl.ds(off, _CHUNK)], idx_v)
            pltpu.async_copy(table_hbm.at[idx_v], rows_v, sem).wait()
            pltpu.sync_copy(rows_v, out_hbm.at[pl.ds(off, _CHUNK)])
            return carry

        lax.fori_loop(0, n_chunks, body, 0)

    return _impl(idx, table)


def kernel(x, table):
    b, l = x.shape
    v, d = table.shape
    n = b * l
    out = _gather_flat(x.reshape(n), table, n, d)
    return out.reshape(b, l, d)


# speedup vs baseline: 1.1131x; 1.0363x over previous
import functools

import jax
import jax.numpy as jnp
from jax import lax
from jax.experimental import pallas as pl
from jax.experimental.pallas import tpu as pltpu
from jax.experimental.pallas import tpu_sc as plsc

_NUM_CORES = 2
_NUM_SUBCORES = 16
_NW = _NUM_CORES * _NUM_SUBCORES
_CHUNK = 640
_NBUF = 4


@functools.partial(jax.jit, static_argnums=(2, 3))
def _gather_flat(idx, table, n, d):
    per_w = n // _NW
    n_chunks = per_w // _CHUNK

    @functools.partial(
        pl.kernel,
        out_type=jax.ShapeDtypeStruct((n, d), jnp.float32),
        mesh=plsc.VectorSubcoreMesh(core_axis_name="c", subcore_axis_name="s"),
        scratch_types=[
            pltpu.VMEM((per_w,), jnp.int32),
            pltpu.VMEM((_NBUF, _CHUNK, d), jnp.float32),
            pltpu.SemaphoreType.DMA((_NBUF,)),
            pltpu.SemaphoreType.DMA((_NBUF,)),
        ],
        compiler_params=pltpu.CompilerParams(use_tc_tiling_on_sc=False),
    )
    def _impl(idx_hbm, table_hbm, out_hbm, idx_v, rows, gsem, wsem):
        wid = lax.axis_index("s") * _NUM_CORES + lax.axis_index("c")
        base = wid * per_w

        pltpu.sync_copy(idx_hbm.at[pl.ds(base, per_w)], idx_v)

        gd = [None] * n_chunks
        wd = [None] * n_chunks

        def start_gather(c):
            b = c % _NBUF
            gd[c] = pltpu.async_copy(
                table_hbm.at[idx_v.at[pl.ds(c * _CHUNK, _CHUNK)]],
                rows.at[b],
                gsem.at[b],
            )

        def start_writeback(c):
            b = c % _NBUF
            gd[c].wait()
            wd[c] = pltpu.async_copy(
                rows.at[b],
                out_hbm.at[pl.ds(base + c * _CHUNK, _CHUNK)],
                wsem.at[b],
            )

        for c in range(n_chunks):
            if c >= _NBUF:
                wd[c - _NBUF].wait()
            start_gather(c)
            j = c - (_NBUF - 1)
            if j >= 0:
                start_writeback(j)
        for j in range(max(0, n_chunks - (_NBUF - 1)), n_chunks):
            start_writeback(j)
        for j in range(max(0, n_chunks - _NBUF), n_chunks):
            wd[j].wait()

    return _impl(idx, table)


def kernel(x, table):
    b, l = x.shape
    v, d = table.shape
    n = b * l
    out = _gather_flat(x.reshape(n), table, n, d)
    return out.reshape(b, l, d)


# speedup vs baseline: 1.9407x; 1.7436x over previous
import functools

import jax
import jax.numpy as jnp
from jax import lax
from jax.experimental import pallas as pl
from jax.experimental.pallas import tpu as pltpu
from jax.experimental.pallas import tpu_sc as plsc

_NUM_CORES = 2
_NUM_SUBCORES = 16
_NW = _NUM_CORES * _NUM_SUBCORES
_CHUNK = 640
_NBUF = 4


@functools.partial(jax.jit, static_argnums=(2, 3))
def _gather_flat(idx, table, n, d):
    per_w = n // _NW
    n_chunks = per_w // _CHUNK

    @functools.partial(
        pl.kernel,
        out_type=jax.ShapeDtypeStruct((n, d), jnp.float32),
        mesh=plsc.VectorSubcoreMesh(core_axis_name="c", subcore_axis_name="s"),
        scratch_types=[
            pltpu.VMEM((per_w,), jnp.int32),
            pltpu.VMEM((_NBUF, _CHUNK, d), jnp.float32),
            pltpu.SemaphoreType.DMA((_NBUF,)),
            pltpu.SemaphoreType.DMA((_NBUF,)),
        ],
        compiler_params=pltpu.CompilerParams(use_tc_tiling_on_sc=False),
    )
    def _impl(idx_hbm, table_hbm, out_hbm, idx_v, rows, gsem, wsem):
        wid = lax.axis_index("s") * _NUM_CORES + lax.axis_index("c")
        base = wid * per_w

        pltpu.sync_copy(idx_hbm.at[pl.ds(base, per_w)], idx_v)

        gd = [None] * n_chunks
        wd = [None] * n_chunks

        def start_gather(c):
            b = c % _NBUF
            gd[c] = pltpu.async_copy(
                table_hbm.at[idx_v.at[pl.ds(c * _CHUNK, _CHUNK)]],
                rows.at[b],
                gsem.at[b],
            )

        def start_writeback(c):
            b = c % _NBUF
            gd[c].wait()
            wd[c] = pltpu.async_copy(
                rows.at[b],
                out_hbm.at[pl.ds(base + c * _CHUNK, _CHUNK)],
                wsem.at[b],
            )

        for c in range(n_chunks):
            if c >= _NBUF:
                wd[c - _NBUF].wait()
            start_gather(c)
            j = c - (_NBUF - 1)
            if j >= 0:
                start_writeback(j)
        for j in range(max(0, n_chunks - (_NBUF - 1)), n_chunks):
            start_writeback(j)
        for j in range(max(0, n_chunks - _NBUF), n_chunks):
            wd[j].wait()

    return _impl(idx, table)


def kernel(x, table):
    b, l = x.shape
    v, d = table.shape
    n = b * l
    out = _gather_flat(x.T.reshape(n), table, n, d)
    return out.reshape(l, b, d).transpose((1, 0, 2))
